# baseline (speedup 1.0000x reference)
import functools

import jax
import jax.numpy as jnp
from jax import lax
from jax.experimental import pallas as pl
from jax.experimental.pallas import tpu as pltpu
from jax.experimental.pallas import tpu_sc as plsc

B, S, D = 4, 4096, 768
N = B * S
NW = 32
ROWS_PER_W = N // NW
C = 16
NCHUNK = ROWS_PER_W // C
LANES = 16
VPR = D // LANES

_mesh = plsc.VectorSubcoreMesh(core_axis_name="c", subcore_axis_name="s")


@functools.partial(
    pl.kernel,
    out_type=jax.ShapeDtypeStruct((N, D), jnp.float32),
    mesh=_mesh,
    scratch_types=[
        pltpu.VMEM((ROWS_PER_W,), jnp.int32),
        pltpu.VMEM((ROWS_PER_W,), jnp.int32),
        pltpu.VMEM((C, D), jnp.float32),
        pltpu.VMEM((C, D), jnp.float32),
        pltpu.VMEM((C, D), jnp.float32),
        pltpu.VMEM((C, D), jnp.float32),
        pltpu.SemaphoreType.DMA,
    ],
)
def _emb_sum(x_hbm, vidx_hbm, tidx_hbm, var_hbm, time_hbm, out_hbm,
             vidx_v, tidx_v, xbuf, vbuf, tbuf, obuf, sem):
    wid = lax.axis_index("s") * 2 + lax.axis_index("c")
    base = wid * ROWS_PER_W
    pltpu.sync_copy(vidx_hbm.at[pl.ds(base, ROWS_PER_W)], vidx_v)
    pltpu.sync_copy(tidx_hbm.at[pl.ds(base, ROWS_PER_W)], tidx_v)

    def chunk_body(g, carry):
        r0 = g * C
        pltpu.sync_copy(x_hbm.at[pl.ds(base + r0, C)], xbuf)
        pltpu.async_copy(var_hbm.at[vidx_v.at[pl.ds(r0, C)]], vbuf, sem).wait()
        pltpu.async_copy(time_hbm.at[tidx_v.at[pl.ds(r0, C)]], tbuf, sem).wait()

        def row_body(r, carry2):
            def vec_body(j, carry3):
                sl = pl.ds(j * LANES, LANES)
                obuf[r, sl] = xbuf[r, sl] + vbuf[r, sl] + tbuf[r, sl]
                return carry3
            return lax.fori_loop(0, VPR, vec_body, carry2)

        lax.fori_loop(0, C, row_body, 0)
        pltpu.sync_copy(obuf, out_hbm.at[pl.ds(base + r0, C)])
        return carry

    lax.fori_loop(0, NCHUNK, chunk_body, 0)


def kernel(x, variable_seq, lead_time_seq, var_table, time_table):
    x2 = x.reshape(N, D)
    vidx = variable_seq.reshape(N).astype(jnp.int32)
    tidx = lead_time_seq.reshape(N).astype(jnp.int32)
    out = _emb_sum(x2, vidx, tidx, var_table, time_table)
    return out.reshape(B, S, D)


# speedup vs baseline: 2.3100x; 2.3100x over previous
import functools

import jax
import jax.numpy as jnp
from jax import lax
from jax.experimental import pallas as pl
from jax.experimental.pallas import tpu as pltpu
from jax.experimental.pallas import tpu_sc as plsc

B, S, D = 4, 4096, 768
N = B * S
NW = 32
ROWS_PER_W = N // NW
C = 16
NCHUNK = ROWS_PER_W // C
NBUF = 3
LANES = 16

_mesh = plsc.VectorSubcoreMesh(core_axis_name="c", subcore_axis_name="s")


@functools.partial(
    pl.kernel,
    out_type=jax.ShapeDtypeStruct((N, D), jnp.float32),
    mesh=_mesh,
    scratch_types=[
        pltpu.VMEM((ROWS_PER_W,), jnp.int32),
        pltpu.VMEM((ROWS_PER_W,), jnp.int32),
        pltpu.VMEM((NBUF, C, D), jnp.float32),
        pltpu.VMEM((NBUF, C, D), jnp.float32),
        pltpu.VMEM((NBUF, C, D), jnp.float32),
        pltpu.SemaphoreType.DMA((NBUF,)),
        pltpu.SemaphoreType.DMA((NBUF,)),
        pltpu.SemaphoreType.DMA((NBUF,)),
        pltpu.SemaphoreType.DMA((NBUF,)),
    ],
)
def _emb_sum(x_hbm, vidx_hbm, tidx_hbm, var_hbm, time_hbm, out_hbm,
             vidx_v, tidx_v, obuf, vbuf, tbuf,
             sem_x, sem_v, sem_t, sem_o):
    wid = lax.axis_index("s") * 2 + lax.axis_index("c")
    base = wid * ROWS_PER_W
    pltpu.sync_copy(vidx_hbm.at[pl.ds(base, ROWS_PER_W)], vidx_v)
    pltpu.sync_copy(tidx_hbm.at[pl.ds(base, ROWS_PER_W)], tidx_v)

    def issue_loads(g):
        s = lax.rem(g, NBUF)
        r0 = g * C
        pltpu.async_copy(x_hbm.at[pl.ds(base + r0, C)], obuf.at[s], sem_x.at[s])
        pltpu.async_copy(var_hbm.at[vidx_v.at[pl.ds(r0, C)]], vbuf.at[s],
                         sem_v.at[s])
        pltpu.async_copy(time_hbm.at[tidx_v.at[pl.ds(r0, C)]], tbuf.at[s],
                         sem_t.at[s])

    def wait_loads(g):
        s = lax.rem(g, NBUF)
        r0 = g * C
        pltpu.make_async_copy(x_hbm.at[pl.ds(base + r0, C)], obuf.at[s],
                              sem_x.at[s]).wait()
        pltpu.make_async_copy(var_hbm.at[vidx_v.at[pl.ds(r0, C)]], vbuf.at[s],
                              sem_v.at[s]).wait()
        pltpu.make_async_copy(time_hbm.at[tidx_v.at[pl.ds(r0, C)]], tbuf.at[s],
                              sem_t.at[s]).wait()

    def wait_store(s):
        pltpu.make_async_copy(obuf.at[s], out_hbm.at[pl.ds(base, C)],
                              sem_o.at[s]).wait()

    issue_loads(jnp.int32(0))
    issue_loads(jnp.int32(1))

    def chunk_body(g, carry):
        s = lax.rem(g, NBUF)
        wait_loads(g)

        def row_body(r, carry2):
            @plsc.parallel_loop(0, D, LANES, unroll=8)
            def _vec(j):
                sl = pl.ds(j, LANES)
                plsc.addupdate(obuf.at[s, r, sl], vbuf[s, r, sl] + tbuf[s, r, sl])
            return carry2

        lax.fori_loop(0, C, row_body, 0)

        pltpu.async_copy(obuf.at[s], out_hbm.at[pl.ds(base + g * C, C)],
                         sem_o.at[s])

        g2 = g + 2

        @pl.when(g2 < NCHUNK)
        def _():
            s2 = lax.rem(g2, NBUF)

            @pl.when(g >= 1)
            def _():
                wait_store(s2)

            issue_loads(g2)

        return carry

    lax.fori_loop(0, NCHUNK, chunk_body, 0)
    wait_store(jnp.int32((NCHUNK - 3) % NBUF))
    wait_store(jnp.int32((NCHUNK - 2) % NBUF))
    wait_store(jnp.int32((NCHUNK - 1) % NBUF))


def kernel(x, variable_seq, lead_time_seq, var_table, time_table):
    x2 = x.reshape(N, D)
    vidx = variable_seq.reshape(N).astype(jnp.int32)
    tidx = lead_time_seq.reshape(N).astype(jnp.int32)
    out = _emb_sum(x2, vidx, tidx, var_table, time_table)
    return out.reshape(B, S, D)
